# baseline (speedup 1.0000x reference)
import functools

import jax
import jax.numpy as jnp
from jax import lax
from jax.experimental import pallas as pl
from jax.experimental.pallas import tpu as pltpu
from jax.experimental.pallas import tpu_sc as plsc

_PHI_LO = 8.5e-8
_PHI_HI = 16.635532

_NW = 32
_CHUNK = 64

_SC_MESH = plsc.VectorSubcoreMesh(core_axis_name="c", subcore_axis_name="s")


def _phi(x):
    x = jnp.clip(x, _PHI_LO, _PHI_HI)
    return -jnp.log(jnp.tanh(x * 0.5))


def _sc_gather(table, idx, e_pad, b):
    rows_pw = e_pad // _NW
    nchunk = rows_pw // _CHUNK

    @functools.partial(
        pl.kernel,
        mesh=_SC_MESH,
        out_type=jax.ShapeDtypeStruct((e_pad, b), jnp.float32),
        scratch_types=[
            pltpu.VMEM((rows_pw,), jnp.int32),
            pltpu.VMEM((_CHUNK, b), jnp.float32),
            pltpu.SemaphoreType.DMA,
        ],
    )
    def k(table_hbm, idx_hbm, out_hbm, idx_v, rows_v, sem):
        wid = lax.axis_index("s") * 2 + lax.axis_index("c")
        base = wid * rows_pw
        pltpu.sync_copy(idx_hbm.at[pl.ds(base, rows_pw)], idx_v)

        @pl.loop(0, nchunk)
        def _(i):
            pltpu.async_copy(
                table_hbm.at[idx_v.at[pl.ds(i * _CHUNK, _CHUNK)]], rows_v, sem
            ).wait()
            pltpu.sync_copy(rows_v, out_hbm.at[pl.ds(base + i * _CHUNK, _CHUNK)])

    return k(table, idx)


_RCHUNK = 8


def _init_body(scale_ref, w_ref, w3_ref, llr_ref, msg_ref):
    scale = scale_ref[0, 0]
    nsteps = w_ref.shape[0] // _RCHUNK

    def step(i, _):
        sl = pl.ds(i * _RCHUNK, _RCHUNK)
        llr = scale * (1.0 + w_ref[sl, :])
        llr_ref[sl, :] = llr
        msg_ref[:, sl, :] = w3_ref[:, sl, :] * llr[None, :, :]
        return 0

    lax.fori_loop(0, nsteps, step, 0, unroll=False)


def _cn_body(x_ref, o_ref):
    nsteps = x_ref.shape[1] // _RCHUNK

    def step(i, _):
        sl = pl.ds(i * _RCHUNK, _RCHUNK)
        x = x_ref[:, sl, :]
        mag = _phi(jnp.abs(x))
        mag_sum = ((mag[0:1] + mag[1:2]) + (mag[2:3] + mag[3:4])) + (
            mag[4:5] + mag[5:6]
        )
        ext = _phi(mag_sum - mag)
        sgn = jnp.where(x < 0.0, -1.0, 1.0)
        stot = ((sgn[0:1] * sgn[1:2]) * (sgn[2:3] * sgn[3:4])) * (
            sgn[4:5] * sgn[5:6]
        )
        o_ref[:, sl, :] = stot * sgn * ext
        return 0

    lax.fori_loop(0, nsteps, step, 0, unroll=False)


def _vn_body(t_ref, llr_ref, w3_ref, msg_ref, loss_ref):
    nsteps = llr_ref.shape[0] // _RCHUNK

    def step(i, acc):
        sl = pl.ds(i * _RCHUNK, _RCHUNK)
        t = t_ref[:, sl, :]
        llr_tot = llr_ref[sl, :] + ((t[0] + t[1]) + t[2])
        mlt = -llr_tot
        sp = jnp.maximum(mlt, 0.0) + jnp.log1p(jnp.exp(-jnp.abs(mlt)))
        msg_ref[:, sl, :] = w3_ref[:, sl, :] * (llr_tot[None, :, :] - t)
        return acc + jnp.sum(sp)

    acc = lax.fori_loop(0, nsteps, step, jnp.float32(0.0), unroll=False)

    @pl.when(pl.program_id(0) == 0)
    def _():
        loss_ref[0, 0] = 0.0

    loss_ref[0, 0] += acc


def _vn_last_body(t_ref, llr_ref, chat_ref, loss_ref):
    nsteps = llr_ref.shape[0] // _RCHUNK

    def step(i, acc):
        sl = pl.ds(i * _RCHUNK, _RCHUNK)
        t = t_ref[:, sl, :]
        llr_tot = llr_ref[sl, :] + ((t[0] + t[1]) + t[2])
        mlt = -llr_tot
        sp = jnp.maximum(mlt, 0.0) + jnp.log1p(jnp.exp(-jnp.abs(mlt)))
        chat_ref[sl, :] = mlt
        return acc + jnp.sum(sp)

    acc = lax.fori_loop(0, nsteps, step, jnp.float32(0.0), unroll=False)

    @pl.when(pl.program_id(0) == 0)
    def _():
        loss_ref[0, 0] = 0.0

    loss_ref[0, 0] += acc


def kernel(w_re, w_im, edge_weights, ebno_db, edge_vn, edge_cn):
    del w_im
    b, n = w_re.shape
    e = edge_vn.shape[0]
    m = e // 6
    align = _NW * _CHUNK * 3
    e_pad = -(-e // align) * align
    n_pad = e_pad // 3
    m_pad = e_pad // 6

    f32 = jnp.float32
    coderate = 1.0 - (m / n)
    scale = (4.0 * 2.0 * coderate) * (10.0 ** (ebno_db.astype(f32) / 10.0))
    scale = jnp.reshape(scale.astype(f32), (1, 1))

    ecn = edge_cn.astype(jnp.int32)
    k_sorted = jnp.argsort(ecn, stable=True).astype(jnp.int32)
    src_vn = (k_sorted % 3) * n_pad + (k_sorted // 3)
    cn_core = src_vn.reshape(m, 6).T
    cn_pad = jnp.arange(6 * (m_pad - m), dtype=jnp.int32) % n
    cn_idx = jnp.concatenate([cn_core, cn_pad.reshape(6, m_pad - m)], axis=1)
    cn_idx = cn_idx.reshape(e_pad)
    inv = jnp.argsort(k_sorted, stable=True).astype(jnp.int32)
    src_cn = (inv % 6) * m_pad + (inv // 6)
    vn_core = src_cn.reshape(n, 3).T
    vn_pad = jnp.arange(3 * (n_pad - n), dtype=jnp.int32) % m
    vn_idx = jnp.concatenate([vn_core, vn_pad.reshape(3, n_pad - n)], axis=1)
    vn_idx = vn_idx.reshape(e_pad)

    w3 = edge_weights.astype(f32).reshape(n, 3).T[:, :, None]
    w_t = w_re.astype(f32).T

    rn = 400
    rm = 200
    grid_n = n // rn
    grid_m = m // rm

    nsplit = 2
    bh = b // nsplit

    init_call = pl.pallas_call(
        _init_body,
        grid=(grid_n,),
        in_specs=[
            pl.BlockSpec(memory_space=pltpu.SMEM),
            pl.BlockSpec((rn, bh), lambda g: (g, 0)),
            pl.BlockSpec((3, rn, 1), lambda g: (0, g, 0)),
        ],
        out_specs=[
            pl.BlockSpec((rn, bh), lambda g: (g, 0)),
            pl.BlockSpec((3, rn, bh), lambda g: (0, g, 0)),
        ],
        out_shape=[
            jax.ShapeDtypeStruct((n, bh), f32),
            jax.ShapeDtypeStruct((3, n_pad, bh), f32),
        ],
    )

    cn_call = pl.pallas_call(
        _cn_body,
        grid=(grid_m,),
        in_specs=[pl.BlockSpec((6, rm, bh), lambda g: (0, g, 0))],
        out_specs=pl.BlockSpec((6, rm, bh), lambda g: (0, g, 0)),
        out_shape=jax.ShapeDtypeStruct((6, m_pad, bh), f32),
    )

    vn_call = pl.pallas_call(
        _vn_body,
        grid=(grid_n,),
        in_specs=[
            pl.BlockSpec((3, rn, bh), lambda g: (0, g, 0)),
            pl.BlockSpec((rn, bh), lambda g: (g, 0)),
            pl.BlockSpec((3, rn, 1), lambda g: (0, g, 0)),
        ],
        out_specs=[
            pl.BlockSpec((3, rn, bh), lambda g: (0, g, 0)),
            pl.BlockSpec(memory_space=pltpu.SMEM),
        ],
        out_shape=[
            jax.ShapeDtypeStruct((3, n_pad, bh), f32),
            jax.ShapeDtypeStruct((1, 1), f32),
        ],
    )

    vn_last_call = pl.pallas_call(
        _vn_last_body,
        grid=(grid_n,),
        in_specs=[
            pl.BlockSpec((3, rn, bh), lambda g: (0, g, 0)),
            pl.BlockSpec((rn, bh), lambda g: (g, 0)),
        ],
        out_specs=[
            pl.BlockSpec((rn, bh), lambda g: (g, 0)),
            pl.BlockSpec(memory_space=pltpu.SMEM),
        ],
        out_shape=[
            jax.ShapeDtypeStruct((n, bh), f32),
            jax.ShapeDtypeStruct((1, 1), f32),
        ],
    )

    num_iter = 5
    halves = range(nsplit)
    llr_t = [None] * nsplit
    msg = [None] * nsplit
    for h in halves:
        llr_t[h], msg[h] = init_call(scale, w_t[:, h * bh : (h + 1) * bh], w3)

    loss = jnp.zeros((), f32)
    chat_t = [None] * nsplit
    for it in range(num_iter):
        msg_cs = [
            _sc_gather(msg[h].reshape(e_pad, bh), cn_idx, e_pad, bh) for h in halves
        ]
        mcn = [cn_call(msg_cs[h].reshape(6, m_pad, bh)) for h in halves]
        t = [
            _sc_gather(mcn[h].reshape(e_pad, bh), vn_idx, e_pad, bh) for h in halves
        ]
        for h in halves:
            if it < num_iter - 1:
                msg[h], lp = vn_call(t[h].reshape(3, n_pad, bh), llr_t[h], w3)
            else:
                chat_t[h], lp = vn_last_call(t[h].reshape(3, n_pad, bh), llr_t[h])
            loss = loss + lp[0, 0]

    loss = loss / (num_iter * b * n)
    c = jnp.zeros((b, n), f32)
    c_hat = jnp.concatenate([ch.T for ch in chat_t], axis=0)
    return (c, c_hat, loss)


# speedup vs baseline: 1.1707x; 1.1707x over previous
import functools

import jax
import jax.numpy as jnp
from jax import lax
from jax.experimental import pallas as pl
from jax.experimental.pallas import tpu as pltpu
from jax.experimental.pallas import tpu_sc as plsc

_PHI_LO = 8.5e-8
_PHI_HI = 16.635532

_NW = 32
_CHUNK = 64

_SC_MESH = plsc.VectorSubcoreMesh(core_axis_name="c", subcore_axis_name="s")


def _phi(x):
    x = jnp.clip(x, _PHI_LO, _PHI_HI)
    return -jnp.log(jnp.tanh(x * 0.5))


def _sc_gather(table, idx, e_pad, b):
    rows_pw = e_pad // _NW
    nchunk = rows_pw // _CHUNK

    npair = nchunk // 2

    @functools.partial(
        pl.kernel,
        mesh=_SC_MESH,
        out_type=jax.ShapeDtypeStruct((e_pad, b), jnp.float32),
        scratch_types=[
            pltpu.VMEM((rows_pw,), jnp.int32),
            pltpu.VMEM((_CHUNK, b), jnp.float32),
            pltpu.SemaphoreType.DMA,
        ],
    )
    def k(table_hbm, idx_hbm, out_hbm, idx_v, rows0, sem0):
        wid = lax.axis_index("s") * 2 + lax.axis_index("c")
        base = wid * rows_pw
        pltpu.sync_copy(idx_hbm.at[pl.ds(base, rows_pw)], idx_v)

        def start(i, buf, sem):
            return pltpu.async_copy(
                table_hbm.at[idx_v.at[pl.ds(i * _CHUNK, _CHUNK)]], buf, sem
            )

        def out(i, buf):
            pltpu.sync_copy(buf, out_hbm.at[pl.ds(base + i * _CHUNK, _CHUNK)])

        @pl.loop(0, nchunk)
        def _(i):
            start(i, rows0, sem0).wait()
            out(i, rows0)

    return k(table, idx)


_RCHUNK = 8


def _init_body(scale_ref, w_ref, w3_ref, llr_ref, msg_ref):
    scale = scale_ref[0, 0]
    nsteps = w_ref.shape[0] // _RCHUNK

    def step(i, _):
        sl = pl.ds(i * _RCHUNK, _RCHUNK)
        llr = scale * (1.0 + w_ref[sl, :])
        llr_ref[sl, :] = llr
        msg_ref[:, sl, :] = w3_ref[:, sl, :] * llr[None, :, :]
        return 0

    lax.fori_loop(0, nsteps, step, 0, unroll=False)


def _cn_body(x_ref, o_ref):
    nsteps = x_ref.shape[1] // _RCHUNK

    def step(i, _):
        sl = pl.ds(i * _RCHUNK, _RCHUNK)
        x = x_ref[:, sl, :]
        t = jnp.tanh(jnp.clip(jnp.abs(x), _PHI_LO, _PHI_HI) * 0.5)
        p = ((t[0:1] * t[1:2]) * (t[2:3] * t[3:4])) * (t[4:5] * t[5:6])
        num = t + p
        den = jnp.maximum(t - p, _PHI_LO * t)
        ext = jnp.log(num / den)
        sgn = jnp.where(x < 0.0, -1.0, 1.0)
        stot = ((sgn[0:1] * sgn[1:2]) * (sgn[2:3] * sgn[3:4])) * (
            sgn[4:5] * sgn[5:6]
        )
        o_ref[:, sl, :] = stot * sgn * ext
        return 0

    lax.fori_loop(0, nsteps, step, 0, unroll=False)


def _vn_body(t_ref, llr_ref, w3_ref, msg_ref, loss_ref):
    nsteps = llr_ref.shape[0] // _RCHUNK

    def step(i, acc):
        sl = pl.ds(i * _RCHUNK, _RCHUNK)
        t = t_ref[:, sl, :]
        llr_tot = llr_ref[sl, :] + ((t[0] + t[1]) + t[2])
        mlt = -llr_tot
        sp = jnp.maximum(mlt, 0.0) + jnp.log1p(jnp.exp(-jnp.abs(mlt)))
        msg_ref[:, sl, :] = w3_ref[:, sl, :] * (llr_tot[None, :, :] - t)
        return acc + jnp.sum(sp)

    acc = lax.fori_loop(0, nsteps, step, jnp.float32(0.0), unroll=False)

    @pl.when(pl.program_id(0) == 0)
    def _():
        loss_ref[0, 0] = 0.0

    loss_ref[0, 0] += acc


def _vn_last_body(t_ref, llr_ref, chat_ref, loss_ref):
    nsteps = llr_ref.shape[0] // _RCHUNK

    def step(i, acc):
        sl = pl.ds(i * _RCHUNK, _RCHUNK)
        t = t_ref[:, sl, :]
        llr_tot = llr_ref[sl, :] + ((t[0] + t[1]) + t[2])
        mlt = -llr_tot
        sp = jnp.maximum(mlt, 0.0) + jnp.log1p(jnp.exp(-jnp.abs(mlt)))
        chat_ref[sl, :] = mlt
        return acc + jnp.sum(sp)

    acc = lax.fori_loop(0, nsteps, step, jnp.float32(0.0), unroll=False)

    @pl.when(pl.program_id(0) == 0)
    def _():
        loss_ref[0, 0] = 0.0

    loss_ref[0, 0] += acc


def kernel(w_re, w_im, edge_weights, ebno_db, edge_vn, edge_cn):
    del w_im
    b, n = w_re.shape
    e = edge_vn.shape[0]
    m = e // 6
    align = _NW * _CHUNK * 3
    e_pad = -(-e // align) * align
    n_pad = e_pad // 3
    m_pad = e_pad // 6

    f32 = jnp.float32
    coderate = 1.0 - (m / n)
    scale = (4.0 * 2.0 * coderate) * (10.0 ** (ebno_db.astype(f32) / 10.0))
    scale = jnp.reshape(scale.astype(f32), (1, 1))

    ecn = edge_cn.astype(jnp.int32)
    k_sorted = jnp.argsort(ecn, stable=True).astype(jnp.int32)
    src_vn = (k_sorted % 3) * n_pad + (k_sorted // 3)
    cn_core = src_vn.reshape(m, 6).T
    cn_pad = jnp.arange(6 * (m_pad - m), dtype=jnp.int32) % n
    cn_idx = jnp.concatenate([cn_core, cn_pad.reshape(6, m_pad - m)], axis=1)
    cn_idx = cn_idx.reshape(e_pad)
    inv = jnp.argsort(k_sorted, stable=True).astype(jnp.int32)
    src_cn = (inv % 6) * m_pad + (inv // 6)
    vn_core = src_cn.reshape(n, 3).T
    vn_pad = jnp.arange(3 * (n_pad - n), dtype=jnp.int32) % m
    vn_idx = jnp.concatenate([vn_core, vn_pad.reshape(3, n_pad - n)], axis=1)
    vn_idx = vn_idx.reshape(e_pad)

    w3 = edge_weights.astype(f32).reshape(n, 3).T[:, :, None]
    w_t = w_re.astype(f32).T

    rn = 400
    rm = 200
    grid_n = n // rn
    grid_m = m // rm

    nsplit = 1
    bh = b // nsplit

    init_call = pl.pallas_call(
        _init_body,
        grid=(grid_n,),
        in_specs=[
            pl.BlockSpec(memory_space=pltpu.SMEM),
            pl.BlockSpec((rn, bh), lambda g: (g, 0)),
            pl.BlockSpec((3, rn, 1), lambda g: (0, g, 0)),
        ],
        out_specs=[
            pl.BlockSpec((rn, bh), lambda g: (g, 0)),
            pl.BlockSpec((3, rn, bh), lambda g: (0, g, 0)),
        ],
        out_shape=[
            jax.ShapeDtypeStruct((n, bh), f32),
            jax.ShapeDtypeStruct((3, n_pad, bh), f32),
        ],
    )

    cn_call = pl.pallas_call(
        _cn_body,
        grid=(grid_m,),
        in_specs=[pl.BlockSpec((6, rm, bh), lambda g: (0, g, 0))],
        out_specs=pl.BlockSpec((6, rm, bh), lambda g: (0, g, 0)),
        out_shape=jax.ShapeDtypeStruct((6, m_pad, bh), f32),
    )

    vn_call = pl.pallas_call(
        _vn_body,
        grid=(grid_n,),
        in_specs=[
            pl.BlockSpec((3, rn, bh), lambda g: (0, g, 0)),
            pl.BlockSpec((rn, bh), lambda g: (g, 0)),
            pl.BlockSpec((3, rn, 1), lambda g: (0, g, 0)),
        ],
        out_specs=[
            pl.BlockSpec((3, rn, bh), lambda g: (0, g, 0)),
            pl.BlockSpec(memory_space=pltpu.SMEM),
        ],
        out_shape=[
            jax.ShapeDtypeStruct((3, n_pad, bh), f32),
            jax.ShapeDtypeStruct((1, 1), f32),
        ],
    )

    vn_last_call = pl.pallas_call(
        _vn_last_body,
        grid=(grid_n,),
        in_specs=[
            pl.BlockSpec((3, rn, bh), lambda g: (0, g, 0)),
            pl.BlockSpec((rn, bh), lambda g: (g, 0)),
        ],
        out_specs=[
            pl.BlockSpec((rn, bh), lambda g: (g, 0)),
            pl.BlockSpec(memory_space=pltpu.SMEM),
        ],
        out_shape=[
            jax.ShapeDtypeStruct((n, bh), f32),
            jax.ShapeDtypeStruct((1, 1), f32),
        ],
    )

    num_iter = 5
    halves = range(nsplit)
    llr_t = [None] * nsplit
    msg = [None] * nsplit
    for h in halves:
        llr_t[h], msg[h] = init_call(scale, w_t[:, h * bh : (h + 1) * bh], w3)

    loss = jnp.zeros((), f32)
    chat_t = [None] * nsplit
    for it in range(num_iter):
        msg_cs = [
            _sc_gather(msg[h].reshape(e_pad, bh), cn_idx, e_pad, bh) for h in halves
        ]
        mcn = [cn_call(msg_cs[h].reshape(6, m_pad, bh)) for h in halves]
        t = [
            _sc_gather(mcn[h].reshape(e_pad, bh), vn_idx, e_pad, bh) for h in halves
        ]
        for h in halves:
            if it < num_iter - 1:
                msg[h], lp = vn_call(t[h].reshape(3, n_pad, bh), llr_t[h], w3)
            else:
                chat_t[h], lp = vn_last_call(t[h].reshape(3, n_pad, bh), llr_t[h])
            loss = loss + lp[0, 0]

    loss = loss / (num_iter * b * n)
    c = jnp.zeros((b, n), f32)
    c_hat = jnp.concatenate([ch.T for ch in chat_t], axis=0)
    return (c, c_hat, loss)


# speedup vs baseline: 1.4013x; 1.1970x over previous
import functools

import jax
import jax.numpy as jnp
from jax import lax
from jax.experimental import pallas as pl
from jax.experimental.pallas import tpu as pltpu
from jax.experimental.pallas import tpu_sc as plsc

_PHI_LO = 8.5e-8
_PHI_HI = 16.635532

_NW = 32
_CHUNK = 64

_SC_MESH = plsc.VectorSubcoreMesh(core_axis_name="c", subcore_axis_name="s")


def _phi(x):
    x = jnp.clip(x, _PHI_LO, _PHI_HI)
    return -jnp.log(jnp.tanh(x * 0.5))


def _sc_gather(table, idx, e_pad, b):
    rows_pw = e_pad // _NW
    nchunk = rows_pw // _CHUNK

    npair = nchunk // 2

    @functools.partial(
        pl.kernel,
        mesh=_SC_MESH,
        out_type=jax.ShapeDtypeStruct((e_pad, b), jnp.float32),
        scratch_types=[
            pltpu.VMEM((rows_pw,), jnp.int32),
            pltpu.VMEM((_CHUNK, b), jnp.float32),
            pltpu.SemaphoreType.DMA,
        ],
    )
    def k(table_hbm, idx_hbm, out_hbm, idx_v, rows0, sem0):
        wid = lax.axis_index("s") * 2 + lax.axis_index("c")
        base = wid * rows_pw
        pltpu.sync_copy(idx_hbm.at[pl.ds(base, rows_pw)], idx_v)

        def start(i, buf, sem):
            return pltpu.async_copy(
                table_hbm.at[idx_v.at[pl.ds(i * _CHUNK, _CHUNK)]], buf, sem
            )

        def out(i, buf):
            pltpu.sync_copy(buf, out_hbm.at[pl.ds(base + i * _CHUNK, _CHUNK)])

        @pl.loop(0, nchunk)
        def _(i):
            start(i, rows0, sem0).wait()
            out(i, rows0)

    return k(table, idx)


_RCHUNK = 8


def _init_body(scale_ref, w_ref, w3_ref, llr_ref, msg_ref):
    scale = scale_ref[0, 0]
    nsteps = w_ref.shape[0] // _RCHUNK

    def step(i, _):
        sl = pl.ds(i * _RCHUNK, _RCHUNK)
        llr = scale * (1.0 + w_ref[sl, :])
        llr_ref[sl, :] = llr
        msg_ref[:, sl, :] = w3_ref[:, sl, :] * llr[None, :, :]
        return 0

    lax.fori_loop(0, nsteps, step, 0, unroll=False)


def _cn_body(x_ref, o_ref):
    nsteps = x_ref.shape[1] // _RCHUNK

    def step(i, _):
        sl = pl.ds(i * _RCHUNK, _RCHUNK)
        x = x_ref[:, sl, :]
        t = jnp.tanh(jnp.clip(jnp.abs(x), _PHI_LO, _PHI_HI) * 0.5)
        p = ((t[0:1] * t[1:2]) * (t[2:3] * t[3:4])) * (t[4:5] * t[5:6])
        num = t + p
        den = jnp.maximum(t - p, _PHI_LO * t)
        ext = jnp.log(num / den)
        sgn = jnp.where(x < 0.0, -1.0, 1.0)
        stot = ((sgn[0:1] * sgn[1:2]) * (sgn[2:3] * sgn[3:4])) * (
            sgn[4:5] * sgn[5:6]
        )
        o_ref[:, sl, :] = stot * sgn * ext
        return 0

    lax.fori_loop(0, nsteps, step, 0, unroll=False)


def _vn_body(t_ref, llr_ref, w3_ref, msg_ref, loss_ref):
    nsteps = llr_ref.shape[0] // _RCHUNK

    def step(i, acc):
        sl = pl.ds(i * _RCHUNK, _RCHUNK)
        t = t_ref[:, sl, :]
        llr_tot = llr_ref[sl, :] + ((t[0] + t[1]) + t[2])
        mlt = -llr_tot
        sp = jnp.maximum(mlt, 0.0) + jnp.log1p(jnp.exp(-jnp.abs(mlt)))
        msg_ref[:, sl, :] = w3_ref[:, sl, :] * (llr_tot[None, :, :] - t)
        return acc + sp

    acc0 = jnp.zeros((_RCHUNK, llr_ref.shape[1]), jnp.float32)
    acc = lax.fori_loop(0, nsteps, step, acc0, unroll=False)

    @pl.when(pl.program_id(0) == 0)
    def _():
        loss_ref[0, 0] = 0.0

    loss_ref[0, 0] += jnp.sum(acc)


def _vn_last_body(t_ref, llr_ref, chat_ref, loss_ref):
    nsteps = llr_ref.shape[0] // _RCHUNK

    def step(i, acc):
        sl = pl.ds(i * _RCHUNK, _RCHUNK)
        t = t_ref[:, sl, :]
        llr_tot = llr_ref[sl, :] + ((t[0] + t[1]) + t[2])
        mlt = -llr_tot
        sp = jnp.maximum(mlt, 0.0) + jnp.log1p(jnp.exp(-jnp.abs(mlt)))
        chat_ref[sl, :] = mlt
        return acc + sp

    acc0 = jnp.zeros((_RCHUNK, llr_ref.shape[1]), jnp.float32)
    acc = lax.fori_loop(0, nsteps, step, acc0, unroll=False)

    @pl.when(pl.program_id(0) == 0)
    def _():
        loss_ref[0, 0] = 0.0

    loss_ref[0, 0] += jnp.sum(acc)


def kernel(w_re, w_im, edge_weights, ebno_db, edge_vn, edge_cn):
    del w_im
    b, n = w_re.shape
    e = edge_vn.shape[0]
    m = e // 6
    align = _NW * _CHUNK * 3
    e_pad = -(-e // align) * align
    n_pad = e_pad // 3
    m_pad = e_pad // 6

    f32 = jnp.float32
    coderate = 1.0 - (m / n)
    scale = (4.0 * 2.0 * coderate) * (10.0 ** (ebno_db.astype(f32) / 10.0))
    scale = jnp.reshape(scale.astype(f32), (1, 1))

    ecn = edge_cn.astype(jnp.int32)
    k_sorted = jnp.argsort(ecn, stable=True).astype(jnp.int32)
    src_vn = (k_sorted % 3) * n_pad + (k_sorted // 3)
    cn_core = src_vn.reshape(m, 6).T
    cn_pad = jnp.arange(6 * (m_pad - m), dtype=jnp.int32) % n
    cn_idx = jnp.concatenate([cn_core, cn_pad.reshape(6, m_pad - m)], axis=1)
    cn_idx = cn_idx.reshape(e_pad)
    inv = jnp.argsort(k_sorted, stable=True).astype(jnp.int32)
    src_cn = (inv % 6) * m_pad + (inv // 6)
    vn_core = src_cn.reshape(n, 3).T
    vn_pad = jnp.arange(3 * (n_pad - n), dtype=jnp.int32) % m
    vn_idx = jnp.concatenate([vn_core, vn_pad.reshape(3, n_pad - n)], axis=1)
    vn_idx = vn_idx.reshape(e_pad)

    w3 = edge_weights.astype(f32).reshape(n, 3).T[:, :, None]
    w_t = w_re.astype(f32).T

    rn = 400
    rm = 200
    grid_n = n // rn
    grid_m = m // rm

    nsplit = 1
    bh = b // nsplit

    init_call = pl.pallas_call(
        _init_body,
        grid=(grid_n,),
        in_specs=[
            pl.BlockSpec(memory_space=pltpu.SMEM),
            pl.BlockSpec((rn, bh), lambda g: (g, 0)),
            pl.BlockSpec((3, rn, 1), lambda g: (0, g, 0)),
        ],
        out_specs=[
            pl.BlockSpec((rn, bh), lambda g: (g, 0)),
            pl.BlockSpec((3, rn, bh), lambda g: (0, g, 0)),
        ],
        out_shape=[
            jax.ShapeDtypeStruct((n, bh), f32),
            jax.ShapeDtypeStruct((3, n_pad, bh), f32),
        ],
    )

    cn_call = pl.pallas_call(
        _cn_body,
        grid=(grid_m,),
        in_specs=[pl.BlockSpec((6, rm, bh), lambda g: (0, g, 0))],
        out_specs=pl.BlockSpec((6, rm, bh), lambda g: (0, g, 0)),
        out_shape=jax.ShapeDtypeStruct((6, m_pad, bh), f32),
    )

    vn_call = pl.pallas_call(
        _vn_body,
        grid=(grid_n,),
        in_specs=[
            pl.BlockSpec((3, rn, bh), lambda g: (0, g, 0)),
            pl.BlockSpec((rn, bh), lambda g: (g, 0)),
            pl.BlockSpec((3, rn, 1), lambda g: (0, g, 0)),
        ],
        out_specs=[
            pl.BlockSpec((3, rn, bh), lambda g: (0, g, 0)),
            pl.BlockSpec(memory_space=pltpu.SMEM),
        ],
        out_shape=[
            jax.ShapeDtypeStruct((3, n_pad, bh), f32),
            jax.ShapeDtypeStruct((1, 1), f32),
        ],
    )

    vn_last_call = pl.pallas_call(
        _vn_last_body,
        grid=(grid_n,),
        in_specs=[
            pl.BlockSpec((3, rn, bh), lambda g: (0, g, 0)),
            pl.BlockSpec((rn, bh), lambda g: (g, 0)),
        ],
        out_specs=[
            pl.BlockSpec((rn, bh), lambda g: (g, 0)),
            pl.BlockSpec(memory_space=pltpu.SMEM),
        ],
        out_shape=[
            jax.ShapeDtypeStruct((n, bh), f32),
            jax.ShapeDtypeStruct((1, 1), f32),
        ],
    )

    num_iter = 5
    halves = range(nsplit)
    llr_t = [None] * nsplit
    msg = [None] * nsplit
    for h in halves:
        llr_t[h], msg[h] = init_call(scale, w_t[:, h * bh : (h + 1) * bh], w3)

    loss = jnp.zeros((), f32)
    chat_t = [None] * nsplit
    for it in range(num_iter):
        msg_cs = [
            _sc_gather(msg[h].reshape(e_pad, bh), cn_idx, e_pad, bh) for h in halves
        ]
        mcn = [cn_call(msg_cs[h].reshape(6, m_pad, bh)) for h in halves]
        t = [
            _sc_gather(mcn[h].reshape(e_pad, bh), vn_idx, e_pad, bh) for h in halves
        ]
        for h in halves:
            if it < num_iter - 1:
                msg[h], lp = vn_call(t[h].reshape(3, n_pad, bh), llr_t[h], w3)
            else:
                chat_t[h], lp = vn_last_call(t[h].reshape(3, n_pad, bh), llr_t[h])
            loss = loss + lp[0, 0]

    loss = loss / (num_iter * b * n)
    c = jnp.zeros((b, n), f32)
    c_hat = jnp.concatenate([ch.T for ch in chat_t], axis=0)
    return (c, c_hat, loss)
